# baseline (speedup 1.0000x reference)
import functools

import jax
import jax.numpy as jnp
from jax import lax
from jax.experimental import pallas as pl
from jax.experimental.pallas import tpu as pltpu
from jax.experimental.pallas import tpu_sc as plsc

D = 128
L = 20
LP = 24
GB = 4
NC = 2
NS = 16
NW = NC * NS
NBUF = 6
LOOKAHEAD = NBUF - 2


@functools.cache
def _make_gather(B):
    RPW = B // NW
    NCHUNK = RPW // GB

    mesh = plsc.VectorSubcoreMesh(core_axis_name="c", subcore_axis_name="s")

    @functools.partial(
        pl.kernel,
        out_type=jax.ShapeDtypeStruct((B * LP, D), jnp.float32),
        mesh=mesh,
        scratch_types=[
            pltpu.VMEM((RPW * LP,), jnp.int32),
            pltpu.VMEM((NBUF, GB * LP, D), jnp.float32),
            pltpu.SemaphoreType.DMA,
            pltpu.SemaphoreType.DMA,
        ],
    )
    def gather_kernel(idx_hbm, table_hbm, out_hbm, idx_v, rows_v, gsem, osem):
        wid = lax.axis_index("s") * NC + lax.axis_index("c")
        row0 = wid * RPW
        pltpu.sync_copy(idx_hbm.at[pl.ds(row0 * LP, RPW * LP)], idx_v)

        def g_start(c):
            b = lax.rem(c, NBUF) if not isinstance(c, int) else c % NBUF
            pltpu.async_copy(
                table_hbm.at[idx_v.at[pl.ds(c * (GB * LP), GB * LP)]],
                rows_v.at[b],
                gsem,
            )

        def g_wait():
            pltpu.make_async_copy(
                table_hbm.at[pl.ds(0, GB * LP)], rows_v.at[0], gsem
            ).wait()

        def o_start(c):
            b = lax.rem(c, NBUF) if not isinstance(c, int) else c % NBUF
            pltpu.async_copy(
                rows_v.at[b],
                out_hbm.at[pl.ds((row0 + c * GB) * LP, GB * LP)],
                osem,
            )

        def o_wait():
            pltpu.make_async_copy(
                rows_v.at[0],
                out_hbm.at[pl.ds(row0 * LP, GB * LP)],
                osem,
            ).wait()

        for c in range(LOOKAHEAD):
            g_start(c)

        @pl.loop(0, NCHUNK + 2)
        def body(c):
            @pl.when(c >= 2)
            def _():
                o_wait()

            @pl.when(c + LOOKAHEAD < NCHUNK)
            def _():
                g_start(c + LOOKAHEAD)

            @pl.when(c < NCHUNK)
            def _():
                g_wait()
                o_start(c)

    return gather_kernel


def kernel(input, weight):
    B, seq = input.shape
    idx = input.astype(jnp.int32)
    idxp = jnp.concatenate(
        [idx, jnp.zeros((B, LP - seq), jnp.int32)], axis=1
    ).reshape(B * LP)
    out = _make_gather(B)(idxp, weight)
    return out.reshape(B, LP, D)[:, :seq, :]


# speedup vs baseline: 8.5285x; 8.5285x over previous
import functools

import jax
import jax.numpy as jnp
from jax import lax
from jax.experimental import pallas as pl
from jax.experimental.pallas import tpu as pltpu
from jax.experimental.pallas import tpu_sc as plsc

D = 128
L = 20
LP = 24
GB = 4
NC = 2
NS = 16
NW = NC * NS
NBUF = 6
LOOKAHEAD = NBUF - 2


@functools.cache
def _make_gather(B):
    RPW = B // NW
    NCHUNK = RPW // GB

    mesh = plsc.VectorSubcoreMesh(core_axis_name="c", subcore_axis_name="s")

    @functools.partial(
        pl.kernel,
        out_type=jax.ShapeDtypeStruct((B * LP, D), jnp.float32),
        mesh=mesh,
        scratch_types=[
            pltpu.VMEM((RPW * LP,), jnp.int32),
            pltpu.VMEM((NBUF, GB * LP, D), jnp.float32),
            pltpu.SemaphoreType.DMA,
            pltpu.SemaphoreType.DMA,
        ],
    )
    def gather_kernel(idx_hbm, table_hbm, out_hbm, idx_v, rows_v, gsem, osem):
        wid = lax.axis_index("s") * NC + lax.axis_index("c")
        row0 = wid * RPW
        pltpu.sync_copy(idx_hbm.at[pl.ds(row0 * LP, RPW * LP)], idx_v)

        def g_start(c):
            b = lax.rem(c, NBUF) if not isinstance(c, int) else c % NBUF
            pltpu.async_copy(
                table_hbm.at[idx_v.at[pl.ds(c * (GB * LP), GB * LP)]],
                rows_v.at[b],
                gsem,
            )

        def g_wait():
            pltpu.make_async_copy(
                table_hbm.at[pl.ds(0, GB * LP)], rows_v.at[0], gsem
            ).wait()

        def o_start(c):
            b = lax.rem(c, NBUF) if not isinstance(c, int) else c % NBUF
            pltpu.async_copy(
                rows_v.at[b],
                out_hbm.at[pl.ds((row0 + c * GB) * LP, GB * LP)],
                osem,
            )

        def o_wait():
            pltpu.make_async_copy(
                rows_v.at[0],
                out_hbm.at[pl.ds(row0 * LP, GB * LP)],
                osem,
            ).wait()

        for c in range(LOOKAHEAD):
            g_start(c)

        @pl.loop(0, NCHUNK + 2)
        def body(c):
            @pl.when(c >= 2)
            def _():
                o_wait()

            @pl.when(c + LOOKAHEAD < NCHUNK)
            def _():
                g_start(c + LOOKAHEAD)

            @pl.when(c < NCHUNK)
            def _():
                g_wait()
                o_start(c)

    return gather_kernel


def kernel(input, weight):
    B, seq = input.shape
    idx = input.astype(jnp.int32)
    idxp = jnp.concatenate([idx, idx[:, : LP - seq]], axis=1).reshape(B * LP)
    out = _make_gather(B)(idxp, weight)
    return out.reshape(B, LP, D)[:, :seq, :]


# speedup vs baseline: 8.9702x; 1.0518x over previous
import functools

import jax
import jax.numpy as jnp
from jax import lax
from jax.experimental import pallas as pl
from jax.experimental.pallas import tpu as pltpu
from jax.experimental.pallas import tpu_sc as plsc

D = 128
L = 20
LP = 24
GB = 4
NC = 2
NS = 16
NW = NC * NS
NBUF = 6
LOOKAHEAD = NBUF - 2


@functools.cache
def _make_gather(B):
    RPW = B // NW
    NCHUNK = RPW // GB

    mesh = plsc.VectorSubcoreMesh(core_axis_name="c", subcore_axis_name="s")

    @functools.partial(
        pl.kernel,
        out_type=jax.ShapeDtypeStruct((B, L, D), jnp.float32),
        mesh=mesh,
        scratch_types=[
            pltpu.VMEM((RPW * LP,), jnp.int32),
            pltpu.VMEM((NBUF, GB * LP, D), jnp.float32),
            pltpu.SemaphoreType.DMA,
            pltpu.SemaphoreType.DMA,
        ],
    )
    def gather_kernel(idx_hbm, table_hbm, out_hbm, idx_v, rows_v, gsem, osem):
        wid = lax.axis_index("s") * NC + lax.axis_index("c")
        row0 = wid * RPW
        pltpu.sync_copy(idx_hbm.at[pl.ds(row0 * LP, RPW * LP)], idx_v)

        def g_start(c):
            b = lax.rem(c, NBUF) if not isinstance(c, int) else c % NBUF
            pltpu.async_copy(
                table_hbm.at[idx_v.at[pl.ds(c * (GB * LP), GB * LP)]],
                rows_v.at[b],
                gsem,
            )

        def g_wait():
            pltpu.make_async_copy(
                table_hbm.at[pl.ds(0, GB * LP)], rows_v.at[0], gsem
            ).wait()

        def o_start(c):
            b = lax.rem(c, NBUF) if not isinstance(c, int) else c % NBUF
            for r in range(GB):
                pltpu.async_copy(
                    rows_v.at[b, pl.ds(r * LP, L), :],
                    out_hbm.at[row0 + c * GB + r],
                    osem,
                )

        def o_wait():
            for r in range(GB):
                pltpu.make_async_copy(
                    rows_v.at[0, pl.ds(0, L), :],
                    out_hbm.at[row0],
                    osem,
                ).wait()

        for c in range(LOOKAHEAD):
            g_start(c)

        @pl.loop(0, NCHUNK + 2)
        def body(c):
            @pl.when(c >= 2)
            def _():
                o_wait()

            @pl.when(c + LOOKAHEAD < NCHUNK)
            def _():
                g_start(c + LOOKAHEAD)

            @pl.when(c < NCHUNK)
            def _():
                g_wait()
                o_start(c)

    return gather_kernel


def kernel(input, weight):
    B, seq = input.shape
    idx = input.astype(jnp.int32)
    idxp = jnp.concatenate([idx, idx[:, : LP - seq]], axis=1).reshape(B * LP)
    out = _make_gather(B)(idxp, weight)
    return out


# speedup vs baseline: 9.4050x; 1.0485x over previous
import functools

import jax
import jax.numpy as jnp
from jax import lax
from jax.experimental import pallas as pl
from jax.experimental.pallas import tpu as pltpu
from jax.experimental.pallas import tpu_sc as plsc

D = 128
L = 20
NC = 2
NS = 16
NW = NC * NS
NBUF = 8
LOOKAHEAD = NBUF - 2


@functools.cache
def _make_gather(B):
    RPW = B // NW
    NCHUNK = RPW

    mesh = plsc.VectorSubcoreMesh(core_axis_name="c", subcore_axis_name="s")

    @functools.partial(
        pl.kernel,
        out_type=jax.ShapeDtypeStruct((B, L, D), jnp.float32),
        mesh=mesh,
        scratch_types=[
            pltpu.VMEM((RPW, L), jnp.int32),
            pltpu.VMEM((NBUF, L, D), jnp.float32),
            pltpu.SemaphoreType.DMA,
            pltpu.SemaphoreType.DMA,
        ],
    )
    def gather_kernel(idx_hbm, table_hbm, out_hbm, idx_v, rows_v, gsem, osem):
        wid = lax.axis_index("s") * NC + lax.axis_index("c")
        row0 = wid * RPW
        pltpu.sync_copy(idx_hbm.at[pl.ds(row0, RPW)], idx_v)

        def g_start(c):
            b = lax.rem(c, NBUF) if not isinstance(c, int) else c % NBUF
            pltpu.async_copy(
                table_hbm.at[idx_v.at[c]],
                rows_v.at[b],
                gsem,
            )

        def g_wait():
            pltpu.make_async_copy(out_hbm.at[row0], rows_v.at[0], gsem).wait()

        def o_start(c):
            b = lax.rem(c, NBUF) if not isinstance(c, int) else c % NBUF
            pltpu.async_copy(rows_v.at[b], out_hbm.at[row0 + c], osem)

        def o_wait():
            pltpu.make_async_copy(rows_v.at[0], out_hbm.at[row0], osem).wait()

        for c in range(LOOKAHEAD):
            g_start(c)

        @pl.loop(0, NCHUNK + 2)
        def body(c):
            @pl.when(c >= 2)
            def _():
                o_wait()

            @pl.when(c + LOOKAHEAD < NCHUNK)
            def _():
                g_start(c + LOOKAHEAD)

            @pl.when(c < NCHUNK)
            def _():
                g_wait()
                o_start(c)

    return gather_kernel


def kernel(input, weight):
    B, seq = input.shape
    out = _make_gather(B)(input.astype(jnp.int32), weight)
    return out
